# Initial kernel scaffold; baseline (speedup 1.0000x reference)
import jax
import jax.numpy as jnp
from jax.experimental import pallas as pl


def kernel(node_features, edge_features, senders, receivers, enc_node_w, enc_node_v, enc_edge_w, enc_edge_v, dec_w, dec_v, b0_ew, b0_ev, b0_nw, b0_nv, b1_ew, b1_ev, b1_nw, b1_nv, b2_ew, b2_ev, b2_nw, b2_nv, b3_ew, b3_ev, b3_nw, b3_nv, b4_ew, b4_ev, b4_nw, b4_nv, b5_ew, b5_ev, b5_nw, b5_nv, b6_ew, b6_ev, b6_nw, b6_nv, b7_ew, b7_ev, b7_nw, b7_nv, b8_ew, b8_ev, b8_nw, b8_nv, b9_ew, b9_ev, b9_nw, b9_nv, b10_ew, b10_ev, b10_nw, b10_nv, b11_ew, b11_ev, b11_nw, b11_nv, b12_ew, b12_ev, b12_nw, b12_nv, b13_ew, b13_ev, b13_nw, b13_nv, b14_ew, b14_ev, b14_nw, b14_nv):
    raise NotImplementedError("write your pallas kernel here")


# speedup vs baseline: 2.1611x; 2.1611x over previous
import jax
import jax.numpy as jnp
from jax.experimental import pallas as pl
from jax.experimental.pallas import tpu as pltpu

_LN_EPS = 1e-5
_N = 1024
_E = 2048
_LAT = 128
_STEPS = 15
_OUT = 3
_ND_PAD = 16
_ED_PAD = 8
_VMEM_LIMIT = 60 * 1024 * 1024


def _bf(x):
    return x.astype(jnp.bfloat16)


def _dot(a, b):
    return jnp.dot(a, b, preferred_element_type=jnp.float32)


def _ln_rows(x, gamma, beta):
    mean = jnp.mean(x, axis=0, keepdims=True)
    diff = x - mean
    var = jnp.mean(diff * diff, axis=0, keepdims=True)
    return diff * jax.lax.rsqrt(var + _LN_EPS) * gamma + beta


def _relu(x):
    return jnp.maximum(x, 0.0)


def _epd_kernel(nf_ref, ef_ref, snd_ref, rcv_ref, rcvc_ref,
                enw1_ref, enw23_ref, env_ref,
                eew1_ref, eew23_ref, eev_ref,
                ewt_ref, evt_ref, nwt1_ref, nwt23_ref, nvt_ref,
                dw12_ref, dw3_ref, dv_ref,
                out_ref, psr_ref, pre_ref, node_buf, edge_buf):
    n, e = _N, _E

    snd = snd_ref[0]
    rcv = rcv_ref[0]
    iota_ne = jax.lax.broadcasted_iota(jnp.int32, (n, e), 0)
    psr_ref[0:n, :] = (jnp.broadcast_to(snd, (n, e)) == iota_ne
                       ).astype(jnp.bfloat16)
    psr_ref[n:2 * n, :] = (jnp.broadcast_to(rcv, (n, e)) == iota_ne
                           ).astype(jnp.bfloat16)
    rcvc = rcvc_ref[0]
    iota_en = jax.lax.broadcasted_iota(jnp.int32, (e, n), 1)
    pre_ref[...] = (jnp.broadcast_to(rcvc, (e, n)) == iota_en
                    ).astype(jnp.bfloat16)

    def enc(x_t, w1, w23_ref, vt):
        h = jax.lax.dot_general(w1, x_t, (((0,), (0,)), ((), ())),
                                preferred_element_type=jnp.float32) + vt[:, 0:1]
        h = _dot(w23_ref[0], _bf(_relu(h))) + vt[:, 1:2]
        h = _dot(w23_ref[1], _bf(_relu(h))) + vt[:, 2:3]
        return _ln_rows(h, vt[:, 3:4], vt[:, 4:5])

    node_buf[...] = enc(nf_ref[0], enw1_ref[...], enw23_ref, env_ref[...])
    edge_buf[...] = enc(ef_ref[0], eew1_ref[...], eew23_ref, eev_ref[...])

    def block(s, carry):
        ev = evt_ref[s]
        node_t = node_buf[...]
        node_b = _bf(node_t)
        n_s = _dot(ewt_ref[s, 0], node_b)
        n_r = _dot(ewt_ref[s, 1], node_b)
        sr = _bf(jnp.concatenate([n_s, n_r], axis=1))
        x = (_dot(sr, psr_ref[...])
             + _dot(ewt_ref[s, 2], _bf(edge_buf[...]))
             + ev[:, 0:1])
        x = _dot(ewt_ref[s, 3], _bf(_relu(x))) + ev[:, 1:2]
        x = _dot(ewt_ref[s, 4], _bf(_relu(x))) + ev[:, 2:3]
        x = _ln_rows(x, ev[:, 3:4], ev[:, 4:5])
        edge_buf[...] += x

        agg_t = _dot(_bf(x), pre_ref[...])
        nv = nvt_ref[s]
        cat = jnp.concatenate([node_b, _bf(agg_t)], axis=0)
        y = _dot(nwt1_ref[s], cat) + nv[:, 0:1]
        y = _dot(nwt23_ref[s, 0], _bf(_relu(y))) + nv[:, 1:2]
        y = _dot(nwt23_ref[s, 1], _bf(_relu(y))) + nv[:, 2:3]
        y = _ln_rows(y, nv[:, 3:4], nv[:, 4:5])
        node_buf[...] = node_t + y
        return carry

    jax.lax.fori_loop(0, _STEPS, block, 0)

    dv = dv_ref[...]
    d = _dot(dw12_ref[0], _bf(node_buf[...])) + dv[:, 0:1]
    d = _dot(dw12_ref[1], _bf(_relu(d))) + dv[:, 1:2]
    out_ref[0] = _dot(dw3_ref[...], _bf(_relu(d))) + dv[0:8, 2:3]


def kernel(node_features, edge_features, senders, receivers,
           enc_node_w, enc_node_v, enc_edge_w, enc_edge_v, dec_w, dec_v,
           b0_ew, b0_ev, b0_nw, b0_nv, b1_ew, b1_ev, b1_nw, b1_nv,
           b2_ew, b2_ev, b2_nw, b2_nv, b3_ew, b3_ev, b3_nw, b3_nv,
           b4_ew, b4_ev, b4_nw, b4_nv, b5_ew, b5_ev, b5_nw, b5_nv,
           b6_ew, b6_ev, b6_nw, b6_nv, b7_ew, b7_ev, b7_nw, b7_nv,
           b8_ew, b8_ev, b8_nw, b8_nv, b9_ew, b9_ev, b9_nw, b9_nv,
           b10_ew, b10_ev, b10_nw, b10_nv, b11_ew, b11_ev, b11_nw, b11_nv,
           b12_ew, b12_ev, b12_nw, b12_nv, b13_ew, b13_ev, b13_nw, b13_nv,
           b14_ew, b14_ev, b14_nw, b14_nv):
    b, n, nd = node_features.shape
    _, e, ed = edge_features.shape
    lp = _LAT

    nf_t = jnp.transpose(
        jnp.pad(node_features, ((0, 0), (0, 0), (0, _ND_PAD - nd))),
        (0, 2, 1)).astype(jnp.bfloat16)
    ef_t = jnp.transpose(
        jnp.pad(edge_features, ((0, 0), (0, 0), (0, _ED_PAD - ed))),
        (0, 2, 1)).astype(jnp.bfloat16)
    snd = senders.reshape(b, 1, e)
    rcv = receivers.reshape(b, 1, e)
    rcvc = receivers.reshape(b, e, 1)

    def t(a):
        return jnp.transpose(a).astype(jnp.bfloat16)

    enw1 = enc_node_w[0:_ND_PAD].astype(jnp.bfloat16)
    enw23 = jnp.stack([t(enc_node_w[_ND_PAD:_ND_PAD + lp]),
                       t(enc_node_w[_ND_PAD + lp:_ND_PAD + 2 * lp])])
    env = jnp.transpose(enc_node_v)
    eew1 = enc_edge_w[0:_ED_PAD].astype(jnp.bfloat16)
    eew23 = jnp.stack([t(enc_edge_w[_ED_PAD:_ED_PAD + lp]),
                       t(enc_edge_w[_ED_PAD + lp:_ED_PAD + 2 * lp])])
    eev = jnp.transpose(enc_edge_v)

    block_args = [
        (b0_ew, b0_ev, b0_nw, b0_nv), (b1_ew, b1_ev, b1_nw, b1_nv),
        (b2_ew, b2_ev, b2_nw, b2_nv), (b3_ew, b3_ev, b3_nw, b3_nv),
        (b4_ew, b4_ev, b4_nw, b4_nv), (b5_ew, b5_ev, b5_nw, b5_nv),
        (b6_ew, b6_ev, b6_nw, b6_nv), (b7_ew, b7_ev, b7_nw, b7_nv),
        (b8_ew, b8_ev, b8_nw, b8_nv), (b9_ew, b9_ev, b9_nw, b9_nv),
        (b10_ew, b10_ev, b10_nw, b10_nv), (b11_ew, b11_ev, b11_nw, b11_nv),
        (b12_ew, b12_ev, b12_nw, b12_nv), (b13_ew, b13_ev, b13_nw, b13_nv),
        (b14_ew, b14_ev, b14_nw, b14_nv),
    ]
    ewt = jnp.stack([
        jnp.stack([t(ew[i * lp:(i + 1) * lp]) for i in range(5)])
        for (ew, _, _, _) in block_args])
    evt = jnp.stack([jnp.transpose(evv)
                     for (_, evv, _, _) in block_args])
    nwt1 = jnp.stack([t(nw[0:2 * lp])
                      for (_, _, nw, _) in block_args])
    nwt23 = jnp.stack([
        jnp.stack([t(nw[2 * lp:3 * lp]), t(nw[3 * lp:4 * lp])])
        for (_, _, nw, _) in block_args])
    nvt = jnp.stack([jnp.transpose(nvv)
                     for (_, _, _, nvv) in block_args])
    dw12 = jnp.stack([t(dec_w[0:lp]), t(dec_w[lp:2 * lp])])
    dw3 = t(dec_w[2 * lp:3 * lp])[0:8]
    dv = jnp.transpose(dec_v)

    out = pl.pallas_call(
        _epd_kernel,
        out_shape=jax.ShapeDtypeStruct((b, 8, n), jnp.float32),
        grid_spec=pltpu.PrefetchScalarGridSpec(
            num_scalar_prefetch=0,
            grid=(b,),
            in_specs=[
                pl.BlockSpec((1, _ND_PAD, n), lambda i: (i, 0, 0)),
                pl.BlockSpec((1, _ED_PAD, e), lambda i: (i, 0, 0)),
                pl.BlockSpec((1, 1, e), lambda i: (i, 0, 0)),
                pl.BlockSpec((1, 1, e), lambda i: (i, 0, 0)),
                pl.BlockSpec((1, e, 1), lambda i: (i, 0, 0)),
                pl.BlockSpec(enw1.shape, lambda i: (0, 0)),
                pl.BlockSpec(enw23.shape, lambda i: (0, 0, 0)),
                pl.BlockSpec(env.shape, lambda i: (0, 0)),
                pl.BlockSpec(eew1.shape, lambda i: (0, 0)),
                pl.BlockSpec(eew23.shape, lambda i: (0, 0, 0)),
                pl.BlockSpec(eev.shape, lambda i: (0, 0)),
                pl.BlockSpec(ewt.shape, lambda i: (0, 0, 0, 0)),
                pl.BlockSpec(evt.shape, lambda i: (0, 0, 0)),
                pl.BlockSpec(nwt1.shape, lambda i: (0, 0, 0)),
                pl.BlockSpec(nwt23.shape, lambda i: (0, 0, 0, 0)),
                pl.BlockSpec(nvt.shape, lambda i: (0, 0, 0)),
                pl.BlockSpec(dw12.shape, lambda i: (0, 0, 0)),
                pl.BlockSpec(dw3.shape, lambda i: (0, 0)),
                pl.BlockSpec(dv.shape, lambda i: (0, 0)),
            ],
            out_specs=pl.BlockSpec((1, 8, n), lambda i: (i, 0, 0)),
            scratch_shapes=[
                pltpu.VMEM((2 * n, e), jnp.bfloat16),
                pltpu.VMEM((e, n), jnp.bfloat16),
                pltpu.VMEM((lp, n), jnp.float32),
                pltpu.VMEM((lp, e), jnp.float32),
            ],
        ),
        compiler_params=pltpu.CompilerParams(
            dimension_semantics=("parallel",),
            vmem_limit_bytes=_VMEM_LIMIT),
    )(nf_t, ef_t, snd, rcv, rcvc, enw1, enw23, env, eew1, eew23, eev,
      ewt, evt, nwt1, nwt23, nvt, dw12, dw3, dv)

    return jnp.transpose(out, (0, 2, 1))[:, :, :_OUT]


# speedup vs baseline: 3.2098x; 1.4853x over previous
import numpy as np

import jax
import jax.numpy as jnp
from jax.experimental import pallas as pl
from jax.experimental.pallas import tpu as pltpu
from jax.sharding import Mesh, PartitionSpec as P

_LN_EPS = 1e-5
_N = 1024
_E = 2048
_LAT = 128
_STEPS = 15
_OUT = 3
_ND_PAD = 16
_ED_PAD = 8
_VMEM_LIMIT = 60 * 1024 * 1024


def _bf(x):
    return x.astype(jnp.bfloat16)


def _dot(a, b):
    return jnp.dot(a, b, preferred_element_type=jnp.float32)


def _ln_rows(x, gamma, beta):
    mean = jnp.mean(x, axis=0, keepdims=True)
    diff = x - mean
    var = jnp.mean(diff * diff, axis=0, keepdims=True)
    return diff * jax.lax.rsqrt(var + _LN_EPS) * gamma + beta


def _relu(x):
    return jnp.maximum(x, 0.0)


def _epd_kernel(nf_ref, ef_ref, snd_ref, rcv_ref, rcvc_ref,
                enw1_ref, enw23_ref, env_ref,
                eew1_ref, eew23_ref, eev_ref,
                ewt_ref, evt_ref, nwt1_ref, nwt23_ref, nvt_ref,
                dw12_ref, dw3_ref, dv_ref,
                out_ref, psr_ref, pre_ref, node_buf, edge_buf):
    n, e = _N, _E

    snd = snd_ref[0]
    rcv = rcv_ref[0]
    iota_ne = jax.lax.broadcasted_iota(jnp.int32, (n, e), 0)
    psr_ref[0:n, :] = (jnp.broadcast_to(snd, (n, e)) == iota_ne
                       ).astype(jnp.bfloat16)
    psr_ref[n:2 * n, :] = (jnp.broadcast_to(rcv, (n, e)) == iota_ne
                           ).astype(jnp.bfloat16)
    rcvc = rcvc_ref[0]
    iota_en = jax.lax.broadcasted_iota(jnp.int32, (e, n), 1)
    pre_ref[...] = (jnp.broadcast_to(rcvc, (e, n)) == iota_en
                    ).astype(jnp.bfloat16)

    def enc(x_t, w1, w23_ref, vt):
        h = jax.lax.dot_general(w1, x_t, (((0,), (0,)), ((), ())),
                                preferred_element_type=jnp.float32) + vt[:, 0:1]
        h = _dot(w23_ref[0], _bf(_relu(h))) + vt[:, 1:2]
        h = _dot(w23_ref[1], _bf(_relu(h))) + vt[:, 2:3]
        return _ln_rows(h, vt[:, 3:4], vt[:, 4:5])

    node_buf[...] = enc(nf_ref[0], enw1_ref[...], enw23_ref, env_ref[...])
    edge_buf[...] = enc(ef_ref[0], eew1_ref[...], eew23_ref, eev_ref[...])

    def block(s, carry):
        ev = evt_ref[s]
        node_t = node_buf[...]
        node_b = _bf(node_t)
        n_s = _dot(ewt_ref[s, 0], node_b)
        n_r = _dot(ewt_ref[s, 1], node_b)
        sr = _bf(jnp.concatenate([n_s, n_r], axis=1))
        x = (_dot(sr, psr_ref[...])
             + _dot(ewt_ref[s, 2], _bf(edge_buf[...]))
             + ev[:, 0:1])
        x = _dot(ewt_ref[s, 3], _bf(_relu(x))) + ev[:, 1:2]
        x = _dot(ewt_ref[s, 4], _bf(_relu(x))) + ev[:, 2:3]
        x = _ln_rows(x, ev[:, 3:4], ev[:, 4:5])
        edge_buf[...] += x

        agg_t = _dot(_bf(x), pre_ref[...])
        nv = nvt_ref[s]
        cat = jnp.concatenate([node_b, _bf(agg_t)], axis=0)
        y = _dot(nwt1_ref[s], cat) + nv[:, 0:1]
        y = _dot(nwt23_ref[s, 0], _bf(_relu(y))) + nv[:, 1:2]
        y = _dot(nwt23_ref[s, 1], _bf(_relu(y))) + nv[:, 2:3]
        y = _ln_rows(y, nv[:, 3:4], nv[:, 4:5])
        node_buf[...] = node_t + y
        return carry

    jax.lax.fori_loop(0, _STEPS, block, 0)

    dv = dv_ref[...]
    d = _dot(dw12_ref[0], _bf(node_buf[...])) + dv[:, 0:1]
    d = _dot(dw12_ref[1], _bf(_relu(d))) + dv[:, 1:2]
    out_ref[0] = _dot(dw3_ref[...], _bf(_relu(d))) + dv[0:8, 2:3]


def kernel(node_features, edge_features, senders, receivers,
           enc_node_w, enc_node_v, enc_edge_w, enc_edge_v, dec_w, dec_v,
           b0_ew, b0_ev, b0_nw, b0_nv, b1_ew, b1_ev, b1_nw, b1_nv,
           b2_ew, b2_ev, b2_nw, b2_nv, b3_ew, b3_ev, b3_nw, b3_nv,
           b4_ew, b4_ev, b4_nw, b4_nv, b5_ew, b5_ev, b5_nw, b5_nv,
           b6_ew, b6_ev, b6_nw, b6_nv, b7_ew, b7_ev, b7_nw, b7_nv,
           b8_ew, b8_ev, b8_nw, b8_nv, b9_ew, b9_ev, b9_nw, b9_nv,
           b10_ew, b10_ev, b10_nw, b10_nv, b11_ew, b11_ev, b11_nw, b11_nv,
           b12_ew, b12_ev, b12_nw, b12_nv, b13_ew, b13_ev, b13_nw, b13_nv,
           b14_ew, b14_ev, b14_nw, b14_nv):
    b, n, nd = node_features.shape
    _, e, ed = edge_features.shape
    lp = _LAT

    nf_t = jnp.transpose(
        jnp.pad(node_features, ((0, 0), (0, 0), (0, _ND_PAD - nd))),
        (0, 2, 1)).astype(jnp.bfloat16)
    ef_t = jnp.transpose(
        jnp.pad(edge_features, ((0, 0), (0, 0), (0, _ED_PAD - ed))),
        (0, 2, 1)).astype(jnp.bfloat16)
    snd = senders.reshape(b, 1, e)
    rcv = receivers.reshape(b, 1, e)
    rcvc = receivers.reshape(b, e, 1)

    def t(a):
        return jnp.transpose(a).astype(jnp.bfloat16)

    enw1 = enc_node_w[0:_ND_PAD].astype(jnp.bfloat16)
    enw23 = jnp.stack([t(enc_node_w[_ND_PAD:_ND_PAD + lp]),
                       t(enc_node_w[_ND_PAD + lp:_ND_PAD + 2 * lp])])
    env = jnp.transpose(enc_node_v)
    eew1 = enc_edge_w[0:_ED_PAD].astype(jnp.bfloat16)
    eew23 = jnp.stack([t(enc_edge_w[_ED_PAD:_ED_PAD + lp]),
                       t(enc_edge_w[_ED_PAD + lp:_ED_PAD + 2 * lp])])
    eev = jnp.transpose(enc_edge_v)

    block_args = [
        (b0_ew, b0_ev, b0_nw, b0_nv), (b1_ew, b1_ev, b1_nw, b1_nv),
        (b2_ew, b2_ev, b2_nw, b2_nv), (b3_ew, b3_ev, b3_nw, b3_nv),
        (b4_ew, b4_ev, b4_nw, b4_nv), (b5_ew, b5_ev, b5_nw, b5_nv),
        (b6_ew, b6_ev, b6_nw, b6_nv), (b7_ew, b7_ev, b7_nw, b7_nv),
        (b8_ew, b8_ev, b8_nw, b8_nv), (b9_ew, b9_ev, b9_nw, b9_nv),
        (b10_ew, b10_ev, b10_nw, b10_nv), (b11_ew, b11_ev, b11_nw, b11_nv),
        (b12_ew, b12_ev, b12_nw, b12_nv), (b13_ew, b13_ev, b13_nw, b13_nv),
        (b14_ew, b14_ev, b14_nw, b14_nv),
    ]
    ewt = jnp.stack([
        jnp.stack([t(ew[i * lp:(i + 1) * lp]) for i in range(5)])
        for (ew, _, _, _) in block_args])
    evt = jnp.stack([jnp.transpose(evv)
                     for (_, evv, _, _) in block_args])
    nwt1 = jnp.stack([t(nw[0:2 * lp])
                      for (_, _, nw, _) in block_args])
    nwt23 = jnp.stack([
        jnp.stack([t(nw[2 * lp:3 * lp]), t(nw[3 * lp:4 * lp])])
        for (_, _, nw, _) in block_args])
    nvt = jnp.stack([jnp.transpose(nvv)
                     for (_, _, _, nvv) in block_args])
    dw12 = jnp.stack([t(dec_w[0:lp]), t(dec_w[lp:2 * lp])])
    dw3 = t(dec_w[2 * lp:3 * lp])[0:8]
    dv = jnp.transpose(dec_v)

    def fwd(nf_t, ef_t, snd, rcv, rcvc, enw1, enw23, env, eew1, eew23, eev,
            ewt, evt, nwt1, nwt23, nvt, dw12, dw3, dv):
        bl = nf_t.shape[0]
        return pl.pallas_call(
            _epd_kernel,
            out_shape=jax.ShapeDtypeStruct((bl, 8, n), jnp.float32),
            grid_spec=pltpu.PrefetchScalarGridSpec(
                num_scalar_prefetch=0,
                grid=(bl,),
                in_specs=[
                    pl.BlockSpec((1, _ND_PAD, n), lambda i: (i, 0, 0)),
                    pl.BlockSpec((1, _ED_PAD, e), lambda i: (i, 0, 0)),
                    pl.BlockSpec((1, 1, e), lambda i: (i, 0, 0)),
                    pl.BlockSpec((1, 1, e), lambda i: (i, 0, 0)),
                    pl.BlockSpec((1, e, 1), lambda i: (i, 0, 0)),
                    pl.BlockSpec(enw1.shape, lambda i: (0, 0)),
                    pl.BlockSpec(enw23.shape, lambda i: (0, 0, 0)),
                    pl.BlockSpec(env.shape, lambda i: (0, 0)),
                    pl.BlockSpec(eew1.shape, lambda i: (0, 0)),
                    pl.BlockSpec(eew23.shape, lambda i: (0, 0, 0)),
                    pl.BlockSpec(eev.shape, lambda i: (0, 0)),
                    pl.BlockSpec(ewt.shape, lambda i: (0, 0, 0, 0)),
                    pl.BlockSpec(evt.shape, lambda i: (0, 0, 0)),
                    pl.BlockSpec(nwt1.shape, lambda i: (0, 0, 0)),
                    pl.BlockSpec(nwt23.shape, lambda i: (0, 0, 0, 0)),
                    pl.BlockSpec(nvt.shape, lambda i: (0, 0, 0)),
                    pl.BlockSpec(dw12.shape, lambda i: (0, 0, 0)),
                    pl.BlockSpec(dw3.shape, lambda i: (0, 0)),
                    pl.BlockSpec(dv.shape, lambda i: (0, 0)),
                ],
                out_specs=pl.BlockSpec((1, 8, n), lambda i: (i, 0, 0)),
                scratch_shapes=[
                    pltpu.VMEM((2 * n, e), jnp.bfloat16),
                    pltpu.VMEM((e, n), jnp.bfloat16),
                    pltpu.VMEM((lp, n), jnp.float32),
                    pltpu.VMEM((lp, e), jnp.float32),
                ],
            ),
            compiler_params=pltpu.CompilerParams(
                dimension_semantics=("parallel",),
                vmem_limit_bytes=_VMEM_LIMIT),
        )(nf_t, ef_t, snd, rcv, rcvc, enw1, enw23, env, eew1, eew23, eev,
          ewt, evt, nwt1, nwt23, nvt, dw12, dw3, dv)

    args = (nf_t, ef_t, snd, rcv, rcvc, enw1, enw23, env, eew1, eew23, eev,
            ewt, evt, nwt1, nwt23, nvt, dw12, dw3, dv)
    devs = jax.devices()
    if len(devs) > 1 and b % len(devs) == 0:
        mesh = Mesh(np.array(devs), ("b",))
        sharded = P("b")
        repl = P()
        out = jax.shard_map(
            fwd, mesh=mesh,
            in_specs=(sharded,) * 5 + (repl,) * 14,
            out_specs=sharded, check_vma=False)(*args)
    else:
        out = fwd(*args)

    return jnp.transpose(out, (0, 2, 1))[:, :, :_OUT]


# speedup vs baseline: 3.2962x; 1.0269x over previous
import functools

import numpy as np

import jax
import jax.numpy as jnp
from jax.experimental import pallas as pl
from jax.experimental.pallas import tpu as pltpu
from jax.sharding import Mesh, PartitionSpec as P

_LN_EPS = 1e-5
_N = 1024
_E = 2048
_LAT = 128
_STEPS = 15
_OUT = 3
_ND_PAD = 16
_ED_PAD = 8
_GPP = 2
_VMEM_LIMIT = 60 * 1024 * 1024


def _bf(x):
    return x.astype(jnp.bfloat16)


def _dot(a, b):
    return jnp.dot(a, b, preferred_element_type=jnp.float32)


def _ln_rows(x, gamma, beta):
    mean = jnp.mean(x, axis=0, keepdims=True)
    diff = x - mean
    var = jnp.mean(diff * diff, axis=0, keepdims=True)
    return diff * jax.lax.rsqrt(var + _LN_EPS) * gamma + beta


def _relu(x):
    return jnp.maximum(x, 0.0)


def _epd_kernel(nf_ref, ef_ref, snd_ref, rcv_ref, rcvc_ref,
                enw1_ref, enw23_ref, env_ref,
                eew1_ref, eew23_ref, eev_ref,
                ewt_ref, evt_ref, nwt1_ref, nwt23_ref, nvt_ref,
                dw12_ref, dw3_ref, dv_ref,
                out_ref, psr_ref, pre_ref, node_buf, edge_buf, *, gpp):
    n, e = _N, _E

    iota_ne = jax.lax.broadcasted_iota(jnp.int32, (n, e), 0)
    iota_en = jax.lax.broadcasted_iota(jnp.int32, (e, n), 1)
    for g in range(gpp):
        snd = snd_ref[g]
        rcv = rcv_ref[g]
        psr_ref[g, 0:n, :] = (jnp.broadcast_to(snd, (n, e)) == iota_ne
                              ).astype(jnp.bfloat16)
        psr_ref[g, n:2 * n, :] = (jnp.broadcast_to(rcv, (n, e)) == iota_ne
                                  ).astype(jnp.bfloat16)
        rcvc = rcvc_ref[g]
        pre_ref[g] = (jnp.broadcast_to(rcvc, (e, n)) == iota_en
                      ).astype(jnp.bfloat16)

    def enc(x_t, w1, w23_ref, vt):
        h = jax.lax.dot_general(w1, x_t, (((0,), (0,)), ((), ())),
                                preferred_element_type=jnp.float32) + vt[:, 0:1]
        h = _dot(w23_ref[0], _bf(_relu(h))) + vt[:, 1:2]
        h = _dot(w23_ref[1], _bf(_relu(h))) + vt[:, 2:3]
        return _ln_rows(h, vt[:, 3:4], vt[:, 4:5])

    for g in range(gpp):
        node_buf[g] = enc(nf_ref[g], enw1_ref[...], enw23_ref, env_ref[...])
        edge_buf[g] = enc(ef_ref[g], eew1_ref[...], eew23_ref, eev_ref[...])

    def block(s, carry):
        ev = evt_ref[s]
        nv = nvt_ref[s]
        for g in range(gpp):
            node_t = node_buf[g]
            node_b = _bf(node_t)
            n_s = _dot(ewt_ref[s, 0], node_b)
            n_r = _dot(ewt_ref[s, 1], node_b)
            sr = _bf(jnp.concatenate([n_s, n_r], axis=1))
            x = (_dot(sr, psr_ref[g])
                 + _dot(ewt_ref[s, 2], _bf(edge_buf[g]))
                 + ev[:, 0:1])
            x = _dot(ewt_ref[s, 3], _bf(_relu(x))) + ev[:, 1:2]
            x = _dot(ewt_ref[s, 4], _bf(_relu(x))) + ev[:, 2:3]
            x = _ln_rows(x, ev[:, 3:4], ev[:, 4:5])
            edge_buf[g] += x

            agg_t = _dot(_bf(x), pre_ref[g])
            cat = jnp.concatenate([node_b, _bf(agg_t)], axis=0)
            y = _dot(nwt1_ref[s], cat) + nv[:, 0:1]
            y = _dot(nwt23_ref[s, 0], _bf(_relu(y))) + nv[:, 1:2]
            y = _dot(nwt23_ref[s, 1], _bf(_relu(y))) + nv[:, 2:3]
            y = _ln_rows(y, nv[:, 3:4], nv[:, 4:5])
            node_buf[g] = node_t + y
        return carry

    jax.lax.fori_loop(0, _STEPS, block, 0)

    dv = dv_ref[...]
    for g in range(gpp):
        d = _dot(dw12_ref[0], _bf(node_buf[g])) + dv[:, 0:1]
        d = _dot(dw12_ref[1], _bf(_relu(d))) + dv[:, 1:2]
        out_ref[g] = _dot(dw3_ref[...], _bf(_relu(d))) + dv[0:8, 2:3]


def kernel(node_features, edge_features, senders, receivers,
           enc_node_w, enc_node_v, enc_edge_w, enc_edge_v, dec_w, dec_v,
           b0_ew, b0_ev, b0_nw, b0_nv, b1_ew, b1_ev, b1_nw, b1_nv,
           b2_ew, b2_ev, b2_nw, b2_nv, b3_ew, b3_ev, b3_nw, b3_nv,
           b4_ew, b4_ev, b4_nw, b4_nv, b5_ew, b5_ev, b5_nw, b5_nv,
           b6_ew, b6_ev, b6_nw, b6_nv, b7_ew, b7_ev, b7_nw, b7_nv,
           b8_ew, b8_ev, b8_nw, b8_nv, b9_ew, b9_ev, b9_nw, b9_nv,
           b10_ew, b10_ev, b10_nw, b10_nv, b11_ew, b11_ev, b11_nw, b11_nv,
           b12_ew, b12_ev, b12_nw, b12_nv, b13_ew, b13_ev, b13_nw, b13_nv,
           b14_ew, b14_ev, b14_nw, b14_nv):
    b, n, nd = node_features.shape
    _, e, ed = edge_features.shape
    lp = _LAT

    nf_t = jnp.transpose(
        jnp.pad(node_features, ((0, 0), (0, 0), (0, _ND_PAD - nd))),
        (0, 2, 1)).astype(jnp.bfloat16)
    ef_t = jnp.transpose(
        jnp.pad(edge_features, ((0, 0), (0, 0), (0, _ED_PAD - ed))),
        (0, 2, 1)).astype(jnp.bfloat16)
    snd = senders.reshape(b, 1, e)
    rcv = receivers.reshape(b, 1, e)
    rcvc = receivers.reshape(b, e, 1)

    def t(a):
        return jnp.transpose(a).astype(jnp.bfloat16)

    enw1 = enc_node_w[0:_ND_PAD].astype(jnp.bfloat16)
    enw23 = jnp.stack([t(enc_node_w[_ND_PAD:_ND_PAD + lp]),
                       t(enc_node_w[_ND_PAD + lp:_ND_PAD + 2 * lp])])
    env = jnp.transpose(enc_node_v)
    eew1 = enc_edge_w[0:_ED_PAD].astype(jnp.bfloat16)
    eew23 = jnp.stack([t(enc_edge_w[_ED_PAD:_ED_PAD + lp]),
                       t(enc_edge_w[_ED_PAD + lp:_ED_PAD + 2 * lp])])
    eev = jnp.transpose(enc_edge_v)

    block_args = [
        (b0_ew, b0_ev, b0_nw, b0_nv), (b1_ew, b1_ev, b1_nw, b1_nv),
        (b2_ew, b2_ev, b2_nw, b2_nv), (b3_ew, b3_ev, b3_nw, b3_nv),
        (b4_ew, b4_ev, b4_nw, b4_nv), (b5_ew, b5_ev, b5_nw, b5_nv),
        (b6_ew, b6_ev, b6_nw, b6_nv), (b7_ew, b7_ev, b7_nw, b7_nv),
        (b8_ew, b8_ev, b8_nw, b8_nv), (b9_ew, b9_ev, b9_nw, b9_nv),
        (b10_ew, b10_ev, b10_nw, b10_nv), (b11_ew, b11_ev, b11_nw, b11_nv),
        (b12_ew, b12_ev, b12_nw, b12_nv), (b13_ew, b13_ev, b13_nw, b13_nv),
        (b14_ew, b14_ev, b14_nw, b14_nv),
    ]
    ewt = jnp.stack([
        jnp.stack([t(ew[i * lp:(i + 1) * lp]) for i in range(5)])
        for (ew, _, _, _) in block_args])
    evt = jnp.stack([jnp.transpose(evv)
                     for (_, evv, _, _) in block_args])
    nwt1 = jnp.stack([t(nw[0:2 * lp])
                      for (_, _, nw, _) in block_args])
    nwt23 = jnp.stack([
        jnp.stack([t(nw[2 * lp:3 * lp]), t(nw[3 * lp:4 * lp])])
        for (_, _, nw, _) in block_args])
    nvt = jnp.stack([jnp.transpose(nvv)
                     for (_, _, _, nvv) in block_args])
    dw12 = jnp.stack([t(dec_w[0:lp]), t(dec_w[lp:2 * lp])])
    dw3 = t(dec_w[2 * lp:3 * lp])[0:8]
    dv = jnp.transpose(dec_v)

    def fwd(nf_t, ef_t, snd, rcv, rcvc, enw1, enw23, env, eew1, eew23, eev,
            ewt, evt, nwt1, nwt23, nvt, dw12, dw3, dv):
        bl = nf_t.shape[0]
        g = _GPP if bl % _GPP == 0 else 1
        return pl.pallas_call(
            functools.partial(_epd_kernel, gpp=g),
            out_shape=jax.ShapeDtypeStruct((bl, 8, n), jnp.float32),
            grid_spec=pltpu.PrefetchScalarGridSpec(
                num_scalar_prefetch=0,
                grid=(bl // g,),
                in_specs=[
                    pl.BlockSpec((g, _ND_PAD, n), lambda i: (i, 0, 0)),
                    pl.BlockSpec((g, _ED_PAD, e), lambda i: (i, 0, 0)),
                    pl.BlockSpec((g, 1, e), lambda i: (i, 0, 0)),
                    pl.BlockSpec((g, 1, e), lambda i: (i, 0, 0)),
                    pl.BlockSpec((g, e, 1), lambda i: (i, 0, 0)),
                    pl.BlockSpec(enw1.shape, lambda i: (0, 0)),
                    pl.BlockSpec(enw23.shape, lambda i: (0, 0, 0)),
                    pl.BlockSpec(env.shape, lambda i: (0, 0)),
                    pl.BlockSpec(eew1.shape, lambda i: (0, 0)),
                    pl.BlockSpec(eew23.shape, lambda i: (0, 0, 0)),
                    pl.BlockSpec(eev.shape, lambda i: (0, 0)),
                    pl.BlockSpec(ewt.shape, lambda i: (0, 0, 0, 0)),
                    pl.BlockSpec(evt.shape, lambda i: (0, 0, 0)),
                    pl.BlockSpec(nwt1.shape, lambda i: (0, 0, 0)),
                    pl.BlockSpec(nwt23.shape, lambda i: (0, 0, 0, 0)),
                    pl.BlockSpec(nvt.shape, lambda i: (0, 0, 0)),
                    pl.BlockSpec(dw12.shape, lambda i: (0, 0, 0)),
                    pl.BlockSpec(dw3.shape, lambda i: (0, 0)),
                    pl.BlockSpec(dv.shape, lambda i: (0, 0)),
                ],
                out_specs=pl.BlockSpec((g, 8, n), lambda i: (i, 0, 0)),
                scratch_shapes=[
                    pltpu.VMEM((g, 2 * n, e), jnp.bfloat16),
                    pltpu.VMEM((g, e, n), jnp.bfloat16),
                    pltpu.VMEM((g, lp, n), jnp.float32),
                    pltpu.VMEM((g, lp, e), jnp.float32),
                ],
            ),
            compiler_params=pltpu.CompilerParams(
                dimension_semantics=("parallel",),
                vmem_limit_bytes=_VMEM_LIMIT),
        )(nf_t, ef_t, snd, rcv, rcvc, enw1, enw23, env, eew1, eew23, eev,
          ewt, evt, nwt1, nwt23, nvt, dw12, dw3, dv)

    args = (nf_t, ef_t, snd, rcv, rcvc, enw1, enw23, env, eew1, eew23, eev,
            ewt, evt, nwt1, nwt23, nvt, dw12, dw3, dv)
    devs = jax.devices()
    if len(devs) > 1 and b % len(devs) == 0:
        mesh = Mesh(np.array(devs), ("b",))
        sharded = P("b")
        repl = P()
        out = jax.shard_map(
            fwd, mesh=mesh,
            in_specs=(sharded,) * 5 + (repl,) * 14,
            out_specs=sharded, check_vma=False)(*args)
    else:
        out = fwd(*args)

    return jnp.transpose(out, (0, 2, 1))[:, :, :_OUT]
